# baseline (speedup 1.0000x reference)
import jax
import jax.numpy as jnp
from jax import lax
from jax.experimental import pallas as pl
from jax.experimental.pallas import tpu as pltpu

VOCAB = 1000
D = 128
B, S = 1024, 20
HB = 512
NSHIFT = 26


def _body(ids_ref, w1_ref, b1_ref, char_ref, shift_ref, shifts_ref,
          w2_ref, b2_ref, out_ref, at_s, shiftT_s, w2b_s):
    @pl.when((pl.program_id(0) == 0) & (pl.program_id(1) == 0))
    def _precompute():
        w1c = w1_ref[0:D]
        at_s[...] = lax.dot_general(
            w1c, char_ref[...], (((0,), (1,)), ((), ())),
            preferred_element_type=jnp.float32).astype(jnp.bfloat16)
        w1s_aug = jnp.concatenate([w1_ref[D:], b1_ref[...]], axis=0)
        se_aug = jnp.concatenate(
            [shift_ref[...], jnp.ones((NSHIFT, 1), jnp.float32)], axis=1)
        ct = lax.dot_general(
            w1s_aug, se_aug, (((0,), (1,)), ((), ())),
            preferred_element_type=jnp.float32)
        ohs = (shifts_ref[...] ==
               lax.broadcasted_iota(jnp.int32, (NSHIFT, B), 0)
               ).astype(jnp.float32)
        full = jnp.dot(ct, ohs, preferred_element_type=jnp.float32)
        shiftT_s[0] = full[:, :HB]
        shiftT_s[1] = full[:, HB:]
        w2b_s[...] = jnp.concatenate(
            [w2_ref[...], b2_ref[...]], axis=0).astype(jnp.bfloat16)

    j = pl.program_id(1)
    ids = ids_ref[0]
    oh = (ids == lax.broadcasted_iota(jnp.int32, (VOCAB, HB), 0)
          ).astype(jnp.bfloat16)
    g = jnp.dot(at_s[...], oh,
                preferred_element_type=jnp.float32)
    h = jnp.maximum(g + shiftT_s[j], 0.0).astype(jnp.bfloat16)
    h_aug = jnp.concatenate([h, jnp.ones((1, HB), jnp.bfloat16)], axis=0)
    out_ref[0] = lax.dot_general(
        w2b_s[...], h_aug, (((0,), (0,)), ((), ())),
        preferred_element_type=jnp.float32)


def kernel(x_chars, x_shifts, char_embed, shift_embed, W1, b1, W2, b2):
    x_chars = x_chars.astype(jnp.int32)
    x_shifts = x_shifts.astype(jnp.int32)

    res = pl.pallas_call(
        _body,
        grid=(S, 2),
        in_specs=[
            pl.BlockSpec((1, 1, HB), lambda s, j: (s, 0, j)),
            pl.BlockSpec((2 * D, D), lambda s, j: (0, 0)),
            pl.BlockSpec((1, D), lambda s, j: (0, 0)),
            pl.BlockSpec((VOCAB, D), lambda s, j: (0, 0)),
            pl.BlockSpec((NSHIFT, D), lambda s, j: (0, 0)),
            pl.BlockSpec((1, B), lambda s, j: (0, 0)),
            pl.BlockSpec((D, VOCAB), lambda s, j: (0, 0)),
            pl.BlockSpec((1, VOCAB), lambda s, j: (0, 0)),
        ],
        out_specs=pl.BlockSpec((1, VOCAB, HB), lambda s, j: (s, 0, j)),
        out_shape=jax.ShapeDtypeStruct((S, VOCAB, B), jnp.float32),
        scratch_shapes=[
            pltpu.VMEM((D, VOCAB), jnp.bfloat16),
            pltpu.VMEM((2, D, HB), jnp.float32),
            pltpu.VMEM((D + 1, VOCAB), jnp.bfloat16),
        ],
    )(x_chars.T.reshape(S, 1, B), W1, b1.reshape(1, D), char_embed,
      shift_embed, x_shifts.reshape(1, B), W2, b2.reshape(1, VOCAB))

    return jnp.transpose(res, (2, 0, 1))


# speedup vs baseline: 1.3385x; 1.3385x over previous
import jax
import jax.numpy as jnp
from jax import lax
from jax.experimental import pallas as pl
from jax.experimental.pallas import tpu as pltpu

VOCAB = 1000
D = 128
B, S = 1024, 20
NSHIFT = 26


def _body(ids_ref, w1_ref, b1_ref, char_ref, shift_ref, shifts_ref,
          w2_ref, b2_ref, out_ref, at_s, shiftT_s, w2b_s):
    @pl.when(pl.program_id(0) == 0)
    def _precompute():
        w1c = w1_ref[0:D]
        at_s[...] = lax.dot_general(
            w1c, char_ref[...], (((0,), (1,)), ((), ())),
            preferred_element_type=jnp.float32).astype(jnp.bfloat16)
        w1s_aug = jnp.concatenate([w1_ref[D:], b1_ref[...]], axis=0)
        se_aug = jnp.concatenate(
            [shift_ref[...], jnp.ones((NSHIFT, 1), jnp.float32)], axis=1)
        ct = lax.dot_general(
            w1s_aug, se_aug, (((0,), (1,)), ((), ())),
            preferred_element_type=jnp.float32)
        ohs = (shifts_ref[...] ==
               lax.broadcasted_iota(jnp.int32, (NSHIFT, B), 0)
               ).astype(jnp.float32)
        shiftT_s[...] = jnp.dot(ct, ohs, preferred_element_type=jnp.float32)
        w2b_s[...] = jnp.concatenate(
            [w2_ref[...], b2_ref[...]], axis=0).astype(jnp.bfloat16)

    ids = ids_ref[0]
    oh = (ids == lax.broadcasted_iota(jnp.int32, (VOCAB, B), 0)
          ).astype(jnp.bfloat16)
    g = jnp.dot(at_s[...], oh,
                preferred_element_type=jnp.float32)
    h = jnp.maximum(g + shiftT_s[...], 0.0).astype(jnp.bfloat16)
    h_aug = jnp.concatenate([h, jnp.ones((1, B), jnp.bfloat16)], axis=0)
    out_ref[0] = lax.dot_general(
        w2b_s[...], h_aug, (((0,), (0,)), ((), ())),
        preferred_element_type=jnp.float32)


def kernel(x_chars, x_shifts, char_embed, shift_embed, W1, b1, W2, b2):
    x_chars = x_chars.astype(jnp.int32)
    x_shifts = x_shifts.astype(jnp.int32)

    res = pl.pallas_call(
        _body,
        grid=(S,),
        in_specs=[
            pl.BlockSpec((1, 1, B), lambda s: (s, 0, 0)),
            pl.BlockSpec((2 * D, D), lambda s: (0, 0)),
            pl.BlockSpec((1, D), lambda s: (0, 0)),
            pl.BlockSpec((VOCAB, D), lambda s: (0, 0)),
            pl.BlockSpec((NSHIFT, D), lambda s: (0, 0)),
            pl.BlockSpec((1, B), lambda s: (0, 0)),
            pl.BlockSpec((D, VOCAB), lambda s: (0, 0)),
            pl.BlockSpec((1, VOCAB), lambda s: (0, 0)),
        ],
        out_specs=pl.BlockSpec((1, VOCAB, B), lambda s: (s, 0, 0)),
        out_shape=jax.ShapeDtypeStruct((S, VOCAB, B), jnp.float32),
        scratch_shapes=[
            pltpu.VMEM((D, VOCAB), jnp.bfloat16),
            pltpu.VMEM((D, B), jnp.float32),
            pltpu.VMEM((D + 1, VOCAB), jnp.bfloat16),
        ],
    )(x_chars.T.reshape(S, 1, B), W1, b1.reshape(1, D), char_embed,
      shift_embed, x_shifts.reshape(1, B), W2, b2.reshape(1, VOCAB))

    return jnp.transpose(res, (2, 0, 1))
